# baseline (speedup 1.0000x reference)
import functools

import jax
import jax.numpy as jnp
from jax import lax
from jax.experimental import pallas as pl
from jax.experimental.pallas import tpu as pltpu
from jax.experimental.pallas import tpu_sc as plsc

_NS = 16
_L = 16


def _rsqrt_newton(x):
    xi = lax.bitcast_convert_type(x, jnp.int32)
    yi = jnp.int32(0x5F3759DF) - lax.shift_right_logical(xi, 1)
    y = lax.bitcast_convert_type(yi, jnp.float32)
    half_x = 0.5 * x
    for _ in range(4):
        y = y * (1.5 - half_x * y * y)
    return y


def _sc_body(B, C, H, W, K, mshift,
             tl_hbm, bl_hbm, br_hbm, tlt_hbm, blt_hbm, brt_hbm,
             out_hbm,
             tags_v, idx6, val6, misc_f, idx_sc, lidx, shared, sem):
    HW = H * W
    CHW = C * HW
    n_chunks = K // _L
    tag_mask = (1 << mshift) - 1

    b = lax.axis_index("s")
    item_base = b * K

    pltpu.sync_copy(tlt_hbm.at[pl.ds(item_base, K)], tags_v.at[0])
    pltpu.sync_copy(blt_hbm.at[pl.ds(item_base, K)], tags_v.at[1])
    pltpu.sync_copy(brt_hbm.at[pl.ds(item_base, K)], tags_v.at[2])

    base_c0 = b * CHW
    for j in range(n_chunks):
        sl = pl.ds(j * _L, _L)
        t = (tags_v[0, sl] & tag_mask) + base_c0
        idx6[0, sl] = t
        idx6[1, sl] = t + HW
        t = tags_v[1, sl] + base_c0
        idx6[2, sl] = t
        idx6[3, sl] = t + HW
        t = tags_v[2, sl] + base_c0
        idx6[4, sl] = t
        idx6[5, sl] = t + HW

    copies = [
        pltpu.async_copy(tl_hbm.at[idx6.at[0]], val6.at[0], sem),
        pltpu.async_copy(tl_hbm.at[idx6.at[1]], val6.at[1], sem),
        pltpu.async_copy(bl_hbm.at[idx6.at[2]], val6.at[2], sem),
        pltpu.async_copy(bl_hbm.at[idx6.at[3]], val6.at[3], sem),
        pltpu.async_copy(br_hbm.at[idx6.at[4]], val6.at[4], sem),
        pltpu.async_copy(br_hbm.at[idx6.at[5]], val6.at[5], sem),
    ]
    for c in copies:
        c.wait()

    if W & (W - 1) == 0:
        w_shift = W.bit_length() - 1

        def split_xy(t):
            return (t & (W - 1)).astype(jnp.float32), \
                   lax.shift_right_logical(t, w_shift).astype(jnp.float32)
    else:
        def split_xy(t):
            return (t % W).astype(jnp.float32), (t // W).astype(jnp.float32)

    acc = jnp.zeros((_L,), jnp.float32)
    n_acc = jnp.zeros((_L,), jnp.float32)
    for j in range(n_chunks):
        sl = pl.ds(j * _L, _L)
        tlt_raw = tags_v[0, sl]
        m = lax.shift_right_logical(tlt_raw, mshift).astype(jnp.float32)
        tl_x, tl_y = split_xy(tlt_raw & tag_mask)
        bl_x, bl_y = split_xy(tags_v[1, sl])
        br_x, br_y = split_xy(tags_v[2, sl])
        tl_xs = tl_x + val6[0, sl]
        tl_ys = tl_y + val6[1, sl]
        bl_xs = bl_x + val6[2, sl]
        bl_ys = bl_y + val6[3, sl]
        br_xs = br_x + val6[4, sl]
        br_ys = br_y + val6[5, sl]
        dx1 = bl_xs - tl_xs
        dy1 = bl_ys - tl_ys
        dx2 = bl_xs - br_xs
        dy2 = bl_ys - br_ys
        w2 = dx2 * dx2 + dy2 * dy2
        h2 = dx1 * dx1 + dy1 * dy1
        ip = dx1 * dx2 + dy1 * dy2
        acc = acc + jnp.abs(ip * _rsqrt_newton(w2 * h2)) * m
        n_acc = n_acc + m

    zvec = jnp.zeros((_L,), jnp.float32)
    misc_f[pl.ds(0, _L)] = acc
    misc_f[pl.ds(_L, _L)] = n_acc
    misc_f[pl.ds(48, _L)] = zvec
    misc_f[pl.ds(64, _L)] = zvec
    misc_f[pl.ds(80, _L)] = zvec
    idx_sc[pl.ds(0, _L)] = jnp.full((_L,), b, jnp.int32)
    idx_sc[pl.ds(_L, _L)] = jnp.full((_L,), _L + b, jnp.int32)

    @pl.when(b == 0)
    def _():
        pltpu.sync_copy(misc_f.at[pl.ds(48, 48)], shared)

    plsc.subcore_barrier()
    pltpu.sync_copy(misc_f.at[pl.ds(0, 2 * _L)], shared.at[idx_sc], add=True)
    plsc.subcore_barrier()

    @pl.when(b == 0)
    def _():
        pltpu.sync_copy(shared, misc_f.at[pl.ds(96, 48)])
        s_vec = misc_f[pl.ds(96, _L)]
        n_vec = misc_f[pl.ds(112, _L)]
        misc_f[pl.ds(32, _L)] = s_vec / (n_vec + 0.0001)
        lidx[pl.ds(0, _L)] = jnp.full((_L,), 2 * _L, jnp.int32)
        lidx[pl.ds(_L, _L)] = jnp.full((_L,), 2 * _L + 8, jnp.int32)
        pltpu.sync_copy(misc_f.at[pl.ds(32, 2 * _L)], shared.at[lidx],
                        add=True)
        pltpu.sync_copy(shared.at[pl.ds(2 * _L, _L)], misc_f.at[pl.ds(144, _L)])
        pltpu.sync_copy(misc_f.at[pl.ds(144, 1)], out_hbm)


def kernel(tl_reg, bl_reg, br_reg, tl_tag, bl_tag, br_tag, mask):
    B, C, H, W = tl_reg.shape
    K = tl_tag.shape[1]
    HW = H * W
    mshift = max(HW.bit_length(), 1)

    tl_f = tl_reg.reshape(-1)
    bl_f = bl_reg.reshape(-1)
    br_f = br_reg.reshape(-1)
    tlt = (tl_tag.astype(jnp.int32)
           | (mask.astype(jnp.int32) << mshift)).reshape(-1)
    blt = bl_tag.astype(jnp.int32).reshape(-1)
    brt = br_tag.astype(jnp.int32).reshape(-1)

    mesh = plsc.VectorSubcoreMesh(core_axis_name="c", subcore_axis_name="s",
                                  num_cores=1, num_subcores=_NS)
    sc = pl.kernel(
        functools.partial(_sc_body, B, C, H, W, K, mshift),
        out_type=jax.ShapeDtypeStruct((1,), jnp.float32),
        mesh=mesh,
        scratch_types=[
            pltpu.VMEM((3, K), jnp.int32),
            pltpu.VMEM((6, K), jnp.int32),
            pltpu.VMEM((6, K), jnp.float32),
            pltpu.VMEM((160,), jnp.float32),
            pltpu.VMEM((2 * _L,), jnp.int32),
            pltpu.VMEM((2 * _L,), jnp.int32),
            pltpu.VMEM_SHARED((48,), jnp.float32),
            pltpu.SemaphoreType.DMA,
        ],
    )
    out = sc(tl_f, bl_f, br_f, tlt, blt, brt)
    return out.reshape(())


# speedup vs baseline: 1.0417x; 1.0417x over previous
import functools

import jax
import jax.numpy as jnp
from jax import lax
from jax.experimental import pallas as pl
from jax.experimental.pallas import tpu as pltpu
from jax.experimental.pallas import tpu_sc as plsc

_NS = 16
_L = 16


def _rsqrt_newton(x):
    xi = lax.bitcast_convert_type(x, jnp.int32)
    yi = jnp.int32(0x5F3759DF) - lax.shift_right_logical(xi, 1)
    y = lax.bitcast_convert_type(yi, jnp.float32)
    half_x = 0.5 * x
    for _ in range(4):
        y = y * (1.5 - half_x * y * y)
    return y


def _sc_body(B, C, H, W, K, mshift,
             tl_hbm, bl_hbm, br_hbm, tags_hbm,
             out_hbm,
             tags_v, idx6, val6, misc_f, idx_sc, lidx, shared, sem):
    HW = H * W
    CHW = C * HW
    n_chunks = K // _L
    tag_mask = (1 << mshift) - 1

    b = lax.axis_index("s")
    item_base = b * K

    pltpu.sync_copy(tags_hbm.at[pl.ds(0, 3), pl.ds(item_base, K)], tags_v)

    base_c0 = b * CHW
    for j in range(n_chunks):
        sl = pl.ds(j * _L, _L)
        t = (tags_v[0, sl] & tag_mask) + base_c0
        idx6[0, sl] = t
        idx6[1, sl] = t + HW
        t = tags_v[1, sl] + base_c0
        idx6[2, sl] = t
        idx6[3, sl] = t + HW
        t = tags_v[2, sl] + base_c0
        idx6[4, sl] = t
        idx6[5, sl] = t + HW

    copies = [
        pltpu.async_copy(tl_hbm.at[idx6.at[0]], val6.at[0], sem),
        pltpu.async_copy(tl_hbm.at[idx6.at[1]], val6.at[1], sem),
        pltpu.async_copy(bl_hbm.at[idx6.at[2]], val6.at[2], sem),
        pltpu.async_copy(bl_hbm.at[idx6.at[3]], val6.at[3], sem),
        pltpu.async_copy(br_hbm.at[idx6.at[4]], val6.at[4], sem),
        pltpu.async_copy(br_hbm.at[idx6.at[5]], val6.at[5], sem),
    ]
    for c in copies:
        c.wait()

    if W & (W - 1) == 0:
        w_shift = W.bit_length() - 1

        def split_xy(t):
            return (t & (W - 1)).astype(jnp.float32), \
                   lax.shift_right_logical(t, w_shift).astype(jnp.float32)
    else:
        def split_xy(t):
            return (t % W).astype(jnp.float32), (t // W).astype(jnp.float32)

    acc = jnp.zeros((_L,), jnp.float32)
    n_acc = jnp.zeros((_L,), jnp.float32)
    for j in range(n_chunks):
        sl = pl.ds(j * _L, _L)
        tlt_raw = tags_v[0, sl]
        m = lax.shift_right_logical(tlt_raw, mshift).astype(jnp.float32)
        tl_x, tl_y = split_xy(tlt_raw & tag_mask)
        bl_x, bl_y = split_xy(tags_v[1, sl])
        br_x, br_y = split_xy(tags_v[2, sl])
        tl_xs = tl_x + val6[0, sl]
        tl_ys = tl_y + val6[1, sl]
        bl_xs = bl_x + val6[2, sl]
        bl_ys = bl_y + val6[3, sl]
        br_xs = br_x + val6[4, sl]
        br_ys = br_y + val6[5, sl]
        dx1 = bl_xs - tl_xs
        dy1 = bl_ys - tl_ys
        dx2 = bl_xs - br_xs
        dy2 = bl_ys - br_ys
        w2 = dx2 * dx2 + dy2 * dy2
        h2 = dx1 * dx1 + dy1 * dy1
        ip = dx1 * dx2 + dy1 * dy2
        acc = acc + jnp.abs(ip * _rsqrt_newton(w2 * h2)) * m
        n_acc = n_acc + m

    zvec = jnp.zeros((_L,), jnp.float32)
    misc_f[pl.ds(0, _L)] = acc
    misc_f[pl.ds(_L, _L)] = n_acc
    misc_f[pl.ds(48, _L)] = zvec
    misc_f[pl.ds(64, _L)] = zvec
    misc_f[pl.ds(80, _L)] = zvec
    idx_sc[pl.ds(0, _L)] = jnp.full((_L,), b, jnp.int32)
    idx_sc[pl.ds(_L, _L)] = jnp.full((_L,), _L + b, jnp.int32)

    @pl.when(b == 0)
    def _():
        pltpu.sync_copy(misc_f.at[pl.ds(48, 48)], shared)

    plsc.subcore_barrier()
    pltpu.sync_copy(misc_f.at[pl.ds(0, 2 * _L)], shared.at[idx_sc], add=True)
    plsc.subcore_barrier()

    @pl.when(b == 0)
    def _():
        pltpu.sync_copy(shared, misc_f.at[pl.ds(96, 48)])
        s_vec = misc_f[pl.ds(96, _L)]
        n_vec = misc_f[pl.ds(112, _L)]
        misc_f[pl.ds(32, _L)] = s_vec / (n_vec + 0.0001)
        lidx[pl.ds(0, _L)] = jnp.full((_L,), 2 * _L, jnp.int32)
        lidx[pl.ds(_L, _L)] = jnp.full((_L,), 2 * _L + 8, jnp.int32)
        pltpu.sync_copy(misc_f.at[pl.ds(32, 2 * _L)], shared.at[lidx],
                        add=True)
        pltpu.sync_copy(shared.at[pl.ds(2 * _L, _L)], misc_f.at[pl.ds(144, _L)])
        pltpu.sync_copy(misc_f.at[pl.ds(144, 1)], out_hbm)


def kernel(tl_reg, bl_reg, br_reg, tl_tag, bl_tag, br_tag, mask):
    B, C, H, W = tl_reg.shape
    K = tl_tag.shape[1]
    HW = H * W
    mshift = max(HW.bit_length(), 1)

    tl_f = tl_reg.reshape(-1)
    bl_f = bl_reg.reshape(-1)
    br_f = br_reg.reshape(-1)
    tags = jnp.stack([
        tl_tag.astype(jnp.int32) | (mask.astype(jnp.int32) << mshift),
        bl_tag.astype(jnp.int32),
        br_tag.astype(jnp.int32),
    ]).reshape(3, B * K)

    mesh = plsc.VectorSubcoreMesh(core_axis_name="c", subcore_axis_name="s",
                                  num_cores=1, num_subcores=_NS)
    sc = pl.kernel(
        functools.partial(_sc_body, B, C, H, W, K, mshift),
        out_type=jax.ShapeDtypeStruct((1,), jnp.float32),
        mesh=mesh,
        scratch_types=[
            pltpu.VMEM((3, K), jnp.int32),
            pltpu.VMEM((6, K), jnp.int32),
            pltpu.VMEM((6, K), jnp.float32),
            pltpu.VMEM((160,), jnp.float32),
            pltpu.VMEM((2 * _L,), jnp.int32),
            pltpu.VMEM((2 * _L,), jnp.int32),
            pltpu.VMEM_SHARED((48,), jnp.float32),
            pltpu.SemaphoreType.DMA,
        ],
    )
    out = sc(tl_f, bl_f, br_f, tags)
    return out.reshape(())


# speedup vs baseline: 1.0426x; 1.0009x over previous
import functools

import jax
import jax.numpy as jnp
from jax import lax
from jax.experimental import pallas as pl
from jax.experimental.pallas import tpu as pltpu
from jax.experimental.pallas import tpu_sc as plsc

_NS = 16
_L = 16


def _rsqrt_newton(x):
    xi = lax.bitcast_convert_type(x, jnp.int32)
    yi = jnp.int32(0x5F3759DF) - lax.shift_right_logical(xi, 1)
    y = lax.bitcast_convert_type(yi, jnp.float32)
    half_x = 0.5 * x
    for _ in range(3):
        y = y * (1.5 - half_x * y * y)
    return y


def _sc_body(B, C, H, W, K, mshift,
             tl_hbm, bl_hbm, br_hbm, tags_hbm,
             out_hbm,
             tags_v, idx6, val6, misc_f, idx_sc, lidx, shared, sem):
    HW = H * W
    CHW = C * HW
    n_chunks = K // _L
    tag_mask = (1 << mshift) - 1

    b = lax.axis_index("s")
    item_base = b * K

    pltpu.sync_copy(tags_hbm.at[pl.ds(0, 3), pl.ds(item_base, K)], tags_v)

    base_c0 = b * CHW
    copies = []
    for r, (hbm, extract) in enumerate([
        (tl_hbm, lambda v: v & tag_mask),
        (bl_hbm, lambda v: v),
        (br_hbm, lambda v: v),
    ]):
        for j in range(n_chunks):
            sl = pl.ds(j * _L, _L)
            t = extract(tags_v[r, sl]) + base_c0
            idx6[2 * r, sl] = t
            idx6[2 * r + 1, sl] = t + HW
        copies.append(pltpu.async_copy(hbm.at[idx6.at[2 * r]],
                                       val6.at[2 * r], sem))
        copies.append(pltpu.async_copy(hbm.at[idx6.at[2 * r + 1]],
                                       val6.at[2 * r + 1], sem))
    for c in copies:
        c.wait()

    if W & (W - 1) == 0:
        w_shift = W.bit_length() - 1

        def split_xy(t):
            return (t & (W - 1)).astype(jnp.float32), \
                   lax.shift_right_logical(t, w_shift).astype(jnp.float32)
    else:
        def split_xy(t):
            return (t % W).astype(jnp.float32), (t // W).astype(jnp.float32)

    acc = jnp.zeros((_L,), jnp.float32)
    n_acc = jnp.zeros((_L,), jnp.float32)
    for j in range(n_chunks):
        sl = pl.ds(j * _L, _L)
        tlt_raw = tags_v[0, sl]
        m = lax.shift_right_logical(tlt_raw, mshift).astype(jnp.float32)
        tl_x, tl_y = split_xy(tlt_raw & tag_mask)
        bl_x, bl_y = split_xy(tags_v[1, sl])
        br_x, br_y = split_xy(tags_v[2, sl])
        tl_xs = tl_x + val6[0, sl]
        tl_ys = tl_y + val6[1, sl]
        bl_xs = bl_x + val6[2, sl]
        bl_ys = bl_y + val6[3, sl]
        br_xs = br_x + val6[4, sl]
        br_ys = br_y + val6[5, sl]
        dx1 = bl_xs - tl_xs
        dy1 = bl_ys - tl_ys
        dx2 = bl_xs - br_xs
        dy2 = bl_ys - br_ys
        w2 = dx2 * dx2 + dy2 * dy2
        h2 = dx1 * dx1 + dy1 * dy1
        ip = dx1 * dx2 + dy1 * dy2
        acc = acc + jnp.abs(ip * _rsqrt_newton(w2 * h2)) * m
        n_acc = n_acc + m

    zvec = jnp.zeros((_L,), jnp.float32)
    misc_f[pl.ds(0, _L)] = acc
    misc_f[pl.ds(_L, _L)] = n_acc
    misc_f[pl.ds(48, _L)] = zvec
    misc_f[pl.ds(64, _L)] = zvec
    misc_f[pl.ds(80, _L)] = zvec
    idx_sc[pl.ds(0, _L)] = jnp.full((_L,), b, jnp.int32)
    idx_sc[pl.ds(_L, _L)] = jnp.full((_L,), _L + b, jnp.int32)

    @pl.when(b == 0)
    def _():
        pltpu.sync_copy(misc_f.at[pl.ds(48, 48)], shared)

    plsc.subcore_barrier()
    pltpu.sync_copy(misc_f.at[pl.ds(0, 2 * _L)], shared.at[idx_sc], add=True)
    plsc.subcore_barrier()

    @pl.when(b == 0)
    def _():
        pltpu.sync_copy(shared, misc_f.at[pl.ds(96, 48)])
        s_vec = misc_f[pl.ds(96, _L)]
        n_vec = misc_f[pl.ds(112, _L)]
        misc_f[pl.ds(32, _L)] = s_vec / (n_vec + 0.0001)
        lidx[pl.ds(0, _L)] = jnp.full((_L,), 2 * _L, jnp.int32)
        lidx[pl.ds(_L, _L)] = jnp.full((_L,), 2 * _L + 8, jnp.int32)
        pltpu.sync_copy(misc_f.at[pl.ds(32, 2 * _L)], shared.at[lidx],
                        add=True)
        pltpu.sync_copy(shared.at[pl.ds(2 * _L, _L)], misc_f.at[pl.ds(144, _L)])
        pltpu.sync_copy(misc_f.at[pl.ds(144, 1)], out_hbm)


def kernel(tl_reg, bl_reg, br_reg, tl_tag, bl_tag, br_tag, mask):
    B, C, H, W = tl_reg.shape
    K = tl_tag.shape[1]
    HW = H * W
    mshift = max(HW.bit_length(), 1)

    tl_f = tl_reg.reshape(-1)
    bl_f = bl_reg.reshape(-1)
    br_f = br_reg.reshape(-1)
    tags = jnp.stack([
        tl_tag.astype(jnp.int32) | (mask.astype(jnp.int32) << mshift),
        bl_tag.astype(jnp.int32),
        br_tag.astype(jnp.int32),
    ]).reshape(3, B * K)

    mesh = plsc.VectorSubcoreMesh(core_axis_name="c", subcore_axis_name="s",
                                  num_cores=1, num_subcores=_NS)
    sc = pl.kernel(
        functools.partial(_sc_body, B, C, H, W, K, mshift),
        out_type=jax.ShapeDtypeStruct((1,), jnp.float32),
        mesh=mesh,
        scratch_types=[
            pltpu.VMEM((3, K), jnp.int32),
            pltpu.VMEM((6, K), jnp.int32),
            pltpu.VMEM((6, K), jnp.float32),
            pltpu.VMEM((160,), jnp.float32),
            pltpu.VMEM((2 * _L,), jnp.int32),
            pltpu.VMEM((2 * _L,), jnp.int32),
            pltpu.VMEM_SHARED((48,), jnp.float32),
            pltpu.SemaphoreType.DMA,
        ],
    )
    out = sc(tl_f, bl_f, br_f, tags)
    return out.reshape(())
